# baseline (speedup 1.0000x reference)
import functools

import jax
import jax.numpy as jnp
from jax import lax
from jax.experimental import pallas as pl
from jax.experimental.pallas import tpu as pltpu
from jax.experimental.pallas import tpu_sc as plsc

_NUM_VECTOR = 8
_LAYER_IDX = 16
_ROWS = 1024
_TAIL = 8
_LANES = 16


def _copy_kernel(x_ref, out_ref):
    out_ref[...] = x_ref[...]


def _patch_kernel(win_ref, new_ref, out_ref):
    out_ref[...] = win_ref[...]
    out_ref[:, _TAIL - 1, :] = new_ref[...]


def _allred(vec, tmp, op):
    for off in (8, 4, 2, 1):
        tmp[pl.ds(0, _LANES)] = vec
        tmp[pl.ds(_LANES, _LANES)] = vec
        vec = op(tmp[pl.ds(0, _LANES)], tmp[pl.ds(off, _LANES)])
    return vec


def _sc_route(xl_hbm, w_hbm, b_hbm, vp_hbm, out_hbm,
              act_v, w_v, b_v, vp_v, new_v, tmp_f, tmp_i, *, nb, hid, nv):
    first = jnp.logical_and(lax.axis_index("c") == 0, lax.axis_index("s") == 0)

    @pl.when(first)
    def _():
        pltpu.sync_copy(xl_hbm, act_v)
        pltpu.sync_copy(w_hbm, w_v)
        pltpu.sync_copy(b_hbm, b_v)
        pltpu.sync_copy(vp_hbm, vp_v)
        lanes = lax.iota(jnp.int32, _LANES)
        nchunk = hid // _LANES
        for b in range(nb):
            svec = jnp.full((_LANES,), -3.0e38, jnp.float32)
            for e in range(nv + 1):

                def dot_body(c, acc, e=e):
                    h0 = c * _LANES
                    av = act_v[b, pl.ds(h0, _LANES)]
                    wv = w_v[e, pl.ds(h0, _LANES)]
                    return acc + av * wv

                acc = lax.fori_loop(0, nchunk, dot_body,
                                    jnp.zeros((_LANES,), jnp.float32))
                s_all = _allred(acc, tmp_f, jnp.add)
                svec = jnp.where(lanes == e, s_all, svec)
            svec = svec + b_v[...]
            m_all = _allred(svec, tmp_f, jnp.maximum)
            cand = jnp.where(svec == m_all, lanes, _LANES)
            idx_all = _allred(cand, tmp_i, jnp.minimum)
            keep = idx_all == nv

            def sel_body(c, carry, idx_all=idx_all, keep=keep):
                h0 = c * _LANES
                av = act_v[b, pl.ds(h0, _LANES)]
                repl = jnp.zeros((_LANES,), jnp.float32)
                for v in range(nv):
                    repl = jnp.where(idx_all == v,
                                     vp_v[v, pl.ds(h0, _LANES)], repl)
                new_v[b, pl.ds(h0, _LANES)] = jnp.where(keep, av, repl)
                return carry

            lax.fori_loop(0, nchunk, sel_body, jnp.int32(0))
        pltpu.sync_copy(new_v, out_hbm)


def kernel(x, vector_pool, gate_W, gate_b):
    B, S, H = x.shape
    vp16 = vector_pool[:, _LAYER_IDX, :]
    xlast = x[:, -1, :]
    x2 = x.reshape(B * S, H)
    nblk = (B * S) // _ROWS

    out1 = pl.pallas_call(
        _copy_kernel,
        grid=(nblk,),
        in_specs=[pl.BlockSpec((_ROWS, H), lambda j: (j, 0))],
        out_specs=pl.BlockSpec((_ROWS, H), lambda j: (j, 0)),
        out_shape=jax.ShapeDtypeStruct((B * S, H), x.dtype),
    )(x2).reshape(B, S, H)

    mesh = plsc.VectorSubcoreMesh(core_axis_name="c", subcore_axis_name="s")
    new_last = pl.kernel(
        functools.partial(_sc_route, nb=B, hid=H, nv=_NUM_VECTOR),
        mesh=mesh,
        out_type=jax.ShapeDtypeStruct((B, H), jnp.float32),
        scratch_types=[
            pltpu.VMEM((B, H), jnp.float32),
            pltpu.VMEM((_NUM_VECTOR + 1, H), jnp.float32),
            pltpu.VMEM((_LANES,), jnp.float32),
            pltpu.VMEM((_NUM_VECTOR, H), jnp.float32),
            pltpu.VMEM((B, H), jnp.float32),
            pltpu.VMEM((2 * _LANES,), jnp.float32),
            pltpu.VMEM((2 * _LANES,), jnp.int32),
        ],
    )(xlast, gate_W.T, jnp.pad(gate_b, (0, _LANES - _NUM_VECTOR - 1),
                               constant_values=-3.0e38), vp16)

    out = pl.pallas_call(
        _patch_kernel,
        grid=(1,),
        in_specs=[
            pl.BlockSpec((B, _TAIL, H), lambda j: (0, S // _TAIL - 1, 0)),
            pl.BlockSpec((B, H), lambda j: (0, 0)),
        ],
        out_specs=pl.BlockSpec((B, _TAIL, H), lambda j: (0, S // _TAIL - 1, 0)),
        out_shape=jax.ShapeDtypeStruct((B, S, H), x.dtype),
        input_output_aliases={0: 0},
    )(out1, new_last)
    return out


# speedup vs baseline: 1.1923x; 1.1923x over previous
import functools

import jax
import jax.numpy as jnp
from jax.experimental import pallas as pl
from jax.experimental.pallas import tpu as pltpu

_NUM_VECTOR = 8
_LAYER_IDX = 16
_ROWS = 1024
_TAIL = 8


def _copy_route_kernel(x_ref, xt_ref, w_ref, b_ref, vp_ref, out_ref, new_ref,
                       *, per_batch):
    j = pl.program_id(0)

    @pl.when(j == 1)
    def _route():
        token_act = xt_ref[:, _TAIL - 1, :]
        scores = jnp.dot(token_act, w_ref[...],
                         preferred_element_type=jnp.float32) + b_ref[...]
        probs = jax.nn.softmax(scores, axis=-1)
        idx = jnp.argmax(probs, axis=-1)
        keep = (idx == _NUM_VECTOR)[:, None]
        nb = token_act.shape[0]
        onehot = (jax.lax.broadcasted_iota(jnp.int32, (nb, _NUM_VECTOR), 1)
                  == jnp.minimum(idx, _NUM_VECTOR - 1)[:, None]).astype(jnp.float32)
        repl = jnp.dot(onehot, vp_ref[...],
                       preferred_element_type=jnp.float32)
        new_ref[...] = jnp.where(keep, token_act, repl)

    out_ref[...] = x_ref[...]

    @pl.when(j % per_batch == per_batch - 1)
    def _patch():
        b = j // per_batch
        out_ref[pl.ds(_ROWS - 1, 1), :] = new_ref[pl.ds(b, 1), :]


def kernel(x, vector_pool, gate_W, gate_b):
    B, S, H = x.shape
    vp16 = vector_pool[:, _LAYER_IDX, :]
    gate_b2 = gate_b.reshape(1, -1)
    x2 = x.reshape(B * S, H)
    nblk = (B * S) // _ROWS
    per_batch = S // _ROWS
    out2 = pl.pallas_call(
        functools.partial(_copy_route_kernel, per_batch=per_batch),
        grid=(nblk,),
        in_specs=[
            pl.BlockSpec((_ROWS, H), lambda j: (j, 0)),
            pl.BlockSpec((B, _TAIL, H), lambda j: (0, S // _TAIL - 1, 0)),
            pl.BlockSpec((H, _NUM_VECTOR + 1), lambda j: (0, 0)),
            pl.BlockSpec((1, _NUM_VECTOR + 1), lambda j: (0, 0)),
            pl.BlockSpec((_NUM_VECTOR, H), lambda j: (0, 0)),
        ],
        out_specs=pl.BlockSpec((_ROWS, H), lambda j: (j, 0)),
        out_shape=jax.ShapeDtypeStruct((B * S, H), x.dtype),
        scratch_shapes=[pltpu.VMEM((B, H), jnp.float32)],
    )(x2, x, gate_W, gate_b2, vp16)
    return out2.reshape(B, S, H)
